# Initial kernel scaffold; baseline (speedup 1.0000x reference)
import jax
import jax.numpy as jnp
from jax.experimental import pallas as pl


def kernel(feat, edge_index, ln1_g, ln1_b, W_head, W_tail, W_ent, attn, g_head, g_tail, ln2_g, ln2_b, W_ff1, b_ff1, W_ff2, b_ff2):
    raise NotImplementedError("write your pallas kernel here")


# speedup vs baseline: 28.7671x; 28.7671x over previous
import functools

import numpy as np
import jax
import jax.numpy as jnp
from jax import lax
from jax.experimental import pallas as pl
from jax.experimental.pallas import tpu as pltpu
from jax.experimental.pallas import tpu_sc as plsc

N = 10000; E = 320000; D = 128; H = 8; DH = 16; HOP = 5; ALPHA = 0.15; DFF = 512
NC = 2; NS = 16; NW = NC * NS
BLK = 128
NBLK = 80
EPT = BLK * NBLK
EPAD = NW * EPT
NACC = 10112
ROWS_PT = NACC // NS
RB = 1000
F32 = jnp.float32

_mesh = plsc.VectorSubcoreMesh(core_axis_name="c", subcore_axis_name="s")


def _wid():
    return lax.axis_index("s") * NC + lax.axis_index("c")


def _zero_acc_stripe(acc, zer_v):
    base = lax.axis_index("s") * ROWS_PT
    for j in range(ROWS_PT // BLK):
        pltpu.sync_copy(zer_v, acc.at[pl.ds(base + j * BLK, BLK)])
    tail = ROWS_PT % BLK
    if tail:
        pltpu.sync_copy(zer_v.at[pl.ds(0, tail)],
                        acc.at[pl.ds(base + (ROWS_PT // BLK) * BLK, tail)])


def _flush_acc_stripe(acc, out_hbm):
    c = lax.axis_index("c")
    base = lax.axis_index("s") * ROWS_PT
    for j in range(ROWS_PT // BLK):
        pltpu.sync_copy(acc.at[pl.ds(base + j * BLK, BLK)],
                        out_hbm.at[pl.ds(c * NACC + base + j * BLK, BLK)])
    tail = ROWS_PT % BLK
    if tail:
        off = (ROWS_PT // BLK) * BLK
        pltpu.sync_copy(acc.at[pl.ds(base + off, tail)],
                        out_hbm.at[pl.ds(c * NACC + base + off, tail)])


_DEGB = 4000


def _degtc_body(d_r, out_r):
    d = d_r[...]
    lo = jnp.bitwise_and(d, 127)
    hi = lax.shift_right_logical(d, 7)
    ior = lax.broadcasted_iota(jnp.int32, (1, 128), 1)
    ohlo = (lo == ior).astype(jnp.bfloat16)
    ohhi = (hi == ior).astype(jnp.bfloat16)
    p = lax.dot_general(ohlo, ohhi, (((0,), (0,)), ((), ())),
                        preferred_element_type=F32)
    i = pl.program_id(0)

    @pl.when(i == 0)
    def _():
        out_r[...] = p

    @pl.when(i > 0)
    def _():
        out_r[...] = out_r[...] + p


def _degtc_call(dst_col):
    return pl.pallas_call(
        _degtc_body,
        grid=(E // _DEGB,),
        in_specs=[pl.BlockSpec((_DEGB, 1), lambda i: (i, 0))],
        out_specs=pl.BlockSpec((128, 128), lambda i: (0, 0)),
        out_shape=jax.ShapeDtypeStruct((128, 128), F32),
    )(dst_col)


@functools.partial(
    pl.kernel,
    out_type=jax.ShapeDtypeStruct((EPAD, 16), F32),
    mesh=_mesh,
    scratch_types=[
        pltpu.VMEM((BLK,), jnp.int32),
        pltpu.VMEM((BLK,), jnp.int32),
        pltpu.VMEM((BLK, 256), F32),
        pltpu.VMEM((BLK, 256), F32),
        pltpu.VMEM((BLK, 16), F32),
        pltpu.VMEM((8, 16), F32),
        pltpu.SemaphoreType.DMA,
    ],
)
def _edge_ac_kernel(s_hbm, t_hbm, g_hbm, sidx_hbm, didxg_hbm, tag_hbm,
                    sblk_v, dblk_v, srow_v, trow_v, tag_v, g_v, sem):
    wid = _wid()
    pltpu.sync_copy(g_hbm, g_v)
    gvec = g_v[0, :]

    lane = lax.broadcasted_iota(jnp.int32, (16,), 0)
    perms = [jnp.bitwise_xor(lane, k) for k in (8, 4, 2, 1)]
    lane8 = jnp.full((16,), 8, jnp.int32)
    msk = lane < H

    def _splat_sum(p):
        for pm in perms:
            p = p + p.at[pm].get(mode="promise_in_bounds",
                                 unique_indices=True)
        return p

    def blk(j, _):
        base = (wid * NBLK + j) * BLK
        pltpu.sync_copy(sidx_hbm.at[pl.ds(base, BLK)], sblk_v)
        pltpu.sync_copy(didxg_hbm.at[pl.ds(base, BLK)], dblk_v)
        pltpu.async_copy(s_hbm.at[sblk_v], srow_v, sem).wait()
        pltpu.async_copy(t_hbm.at[dblk_v], trow_v, sem).wait()

        def edge(e, _2):
            sext = srow_v[e, pl.ds(128, 16)]
            text = trow_v[e, pl.ds(128, 16)]
            gate = 1.0 / (1.0 + jnp.exp(-(sext + text)))
            lgs = text.at[lane8].get(mode="promise_in_bounds") * (1.0 / DH)
            svec = jnp.zeros((16,), F32)
            for h in range(H):
                p = srow_v[e, pl.ds(h * 16, 16)] * trow_v[e, pl.ds(h * 16, 16)]
                sco = _splat_sum(p) * lgs
                svec = jnp.where(lane == h, sco, svec)
            t = jnp.exp(svec - gvec) * gate
            tag_v[e, :] = jnp.where(msk, t, 0.0)
            return 0
        lax.fori_loop(0, BLK, edge, 0)
        pltpu.sync_copy(tag_v, tag_hbm.at[pl.ds(base, BLK)])
        return 0
    lax.fori_loop(0, NBLK, blk, 0)


@functools.partial(
    pl.kernel,
    out_type=jax.ShapeDtypeStruct((NC * NACC, D), F32),
    mesh=_mesh,
    scratch_types=[
        pltpu.VMEM((BLK,), jnp.int32),
        pltpu.VMEM((BLK,), jnp.int32),
        pltpu.VMEM((BLK, D), F32),
        pltpu.VMEM((BLK, 16), F32),
        pltpu.VMEM_SHARED((NACC, D), F32),
        pltpu.SemaphoreType.DMA,
    ],
)
def _hop_kernel(f_hbm, tag_hbm, sidx_hbm, didx_hbm, out_hbm,
                sblk_v, dblk_v, rows_v, tag_v, acc, sem):
    wid = _wid()
    z16 = jnp.zeros((16,), F32)

    def fill(i, _):
        for hh in range(D // 16):
            rows_v[i, pl.ds(hh * 16, 16)] = z16
        return 0
    lax.fori_loop(0, BLK, fill, 0)
    _zero_acc_stripe(acc, rows_v)
    plsc.subcore_barrier()

    lane = lax.broadcasted_iota(jnp.int32, (16,), 0)
    hsplats = [lane * 0 + h for h in range(H)]

    def blk(j, _):
        base = (wid * NBLK + j) * BLK
        pltpu.sync_copy(sidx_hbm.at[pl.ds(base, BLK)], sblk_v)
        pltpu.sync_copy(didx_hbm.at[pl.ds(base, BLK)], dblk_v)
        pltpu.async_copy(f_hbm.at[sblk_v], rows_v, sem).wait()
        pltpu.sync_copy(tag_hbm.at[pl.ds(base, BLK)], tag_v)

        def edge(e, _2):
            trow = tag_v[e, pl.ds(0, 16)]
            for h in range(H):
                sp = trow.at[hsplats[h]].get(mode="promise_in_bounds")
                rows_v[e, pl.ds(h * 16, 16)] = rows_v[e, pl.ds(h * 16, 16)] * sp
            return 0
        lax.fori_loop(0, BLK, edge, 0)
        pltpu.sync_copy(rows_v, acc.at[dblk_v], add=True)
        return 0
    lax.fori_loop(0, NBLK, blk, 0)
    plsc.subcore_barrier()
    _flush_acc_stripe(acc, out_hbm)


def _prep_body(feat_r, g1_r, b1_r, wh_r, wt_r, we_r, attn_r, ghr_r, gtr_r,
               mg_r, dg_r,
               fh_o, fta_o, fe_o, ghp_o, gtp_o, lg_o, msf_o, mst_o):
    xf = feat_r[...]
    mu = jnp.mean(xf, axis=1, keepdims=True)
    xc = xf - mu
    var = jnp.mean(xc * xc, axis=1, keepdims=True)
    x = xc * lax.rsqrt(var + 1e-5) * g1_r[...] + b1_r[...]
    dn = (((1,), (1,)), ((), ()))
    fh = lax.dot_general(x, wh_r[...], dn, preferred_element_type=F32)
    ft = lax.dot_general(x, wt_r[...], dn, preferred_element_type=F32)
    fe = lax.dot_general(x, we_r[...], dn, preferred_element_type=F32)
    fta = ft * attn_r[...]
    fh_o[...] = fh
    fta_o[...] = fta
    fe_o[...] = fe
    dng = (((1,), (0,)), ((), ()))
    ghp_o[...] = lax.dot_general(fh * ghr_r[...], mg_r[...], dng,
                                 preferred_element_type=F32)
    gtp_o[...] = lax.dot_general(ft * gtr_r[...], mg_r[...], dng,
                                 preferred_element_type=F32)
    lg = jnp.log(jnp.maximum(dg_r[...], 1.0))
    lg_o[...] = lg
    fhsq = lax.dot_general(fh * fh, mg_r[...], dng, preferred_element_type=F32)
    ftasq = lax.dot_general(fta * fta, mg_r[...], dng,
                            preferred_element_type=F32)
    ftasq = ftasq * (lg[:, :1] * lg[:, :1])
    bmax_f = jnp.max(fhsq, axis=0, keepdims=True)
    bmax_t = jnp.max(ftasq, axis=0, keepdims=True)
    i = pl.program_id(0)

    @pl.when(i == 0)
    def _():
        msf_o[...] = bmax_f
        mst_o[...] = bmax_t

    @pl.when(i > 0)
    def _():
        msf_o[...] = jnp.maximum(msf_o[...], bmax_f)
        mst_o[...] = jnp.maximum(mst_o[...], bmax_t)


def _prep_call(feat, g1, b1, wh, wt, we, attnrep, ghrep, gtrep, mg, dg):
    row = lambda i: (i, 0)
    full = lambda i: (0, 0)
    return pl.pallas_call(
        _prep_body,
        grid=(N // RB,),
        in_specs=[
            pl.BlockSpec((RB, D), row),
            pl.BlockSpec((1, D), full), pl.BlockSpec((1, D), full),
            pl.BlockSpec((D, D), full), pl.BlockSpec((D, D), full),
            pl.BlockSpec((D, D), full),
            pl.BlockSpec((1, D), full), pl.BlockSpec((1, D), full),
            pl.BlockSpec((1, D), full), pl.BlockSpec((D, D), full),
            pl.BlockSpec((RB, 16), row),
        ],
        out_specs=[
            pl.BlockSpec((RB, D), row), pl.BlockSpec((RB, D), row),
            pl.BlockSpec((RB, D), row), pl.BlockSpec((RB, D), row),
            pl.BlockSpec((RB, D), row), pl.BlockSpec((RB, 16), row),
            pl.BlockSpec((1, D), full), pl.BlockSpec((1, D), full),
        ],
        out_shape=[
            jax.ShapeDtypeStruct((N, D), F32), jax.ShapeDtypeStruct((N, D), F32),
            jax.ShapeDtypeStruct((N, D), F32), jax.ShapeDtypeStruct((N, D), F32),
            jax.ShapeDtypeStruct((N, D), F32), jax.ShapeDtypeStruct((N, 16), F32),
            jax.ShapeDtypeStruct((1, D), F32), jax.ShapeDtypeStruct((1, D), F32),
        ],
    )(feat, g1, b1, wh, wt, we, attnrep, ghrep, gtrep, mg, dg)


def _rinv_body(s0_r, s1_r, out_r):
    den = s0_r[...] + s1_r[...]
    out_r[...] = (1.0 - ALPHA) / jnp.maximum(den, 1e-30)


def _rinv_call(s0, s1):
    row = lambda i: (i, 0)
    return pl.pallas_call(
        _rinv_body,
        grid=(N // RB,),
        in_specs=[pl.BlockSpec((RB, D), row), pl.BlockSpec((RB, D), row)],
        out_specs=pl.BlockSpec((RB, D), row),
        out_shape=jax.ShapeDtypeStruct((N, D), F32),
    )(s0, s1)


def _comb_body(h0_r, h1_r, ri_r, fe_r, out_r):
    out_r[...] = ri_r[...] * (h0_r[...] + h1_r[...]) + ALPHA * fe_r[...]


def _comb_call(h0, h1, rinvrep, fe):
    row = lambda i: (i, 0)
    return pl.pallas_call(
        _comb_body,
        grid=(N // RB,),
        in_specs=[pl.BlockSpec((RB, D), row)] * 4,
        out_specs=pl.BlockSpec((RB, D), row),
        out_shape=jax.ShapeDtypeStruct((N, D), F32),
    )(h0, h1, rinvrep, fe)


def _final_body(f_r, feat_r, g2_r, b2_r, w1_r, bf1_r, w2_r, bf2_r, out_r):
    rst = f_r[...] + feat_r[...]
    mu = jnp.mean(rst, axis=1, keepdims=True)
    xc = rst - mu
    var = jnp.mean(xc * xc, axis=1, keepdims=True)
    y = xc * lax.rsqrt(var + 1e-5) * g2_r[...] + b2_r[...]
    dn = (((1,), (1,)), ((), ()))
    h1 = jnp.maximum(
        lax.dot_general(y, w1_r[...], dn, preferred_element_type=F32)
        + bf1_r[...], 0.0)
    out_r[...] = (lax.dot_general(h1, w2_r[...], dn,
                                  preferred_element_type=F32)
                  + bf2_r[...] + rst)


def _final_call(f, feat, g2, b2, w1, bf1, w2, bf2):
    row = lambda i: (i, 0)
    full = lambda i: (0, 0)
    return pl.pallas_call(
        _final_body,
        grid=(N // RB,),
        in_specs=[
            pl.BlockSpec((RB, D), row), pl.BlockSpec((RB, D), row),
            pl.BlockSpec((1, D), full), pl.BlockSpec((1, D), full),
            pl.BlockSpec((DFF, D), full), pl.BlockSpec((1, DFF), full),
            pl.BlockSpec((D, DFF), full), pl.BlockSpec((1, D), full),
        ],
        out_specs=pl.BlockSpec((RB, D), row),
        out_shape=jax.ShapeDtypeStruct((N, D), F32),
    )(f, feat, g2, b2, w1, bf1, w2, bf2)


_MG = np.zeros((D, D), np.float32)
for _j in range(D):
    _MG[_j, _j // DH] = 1.0


def kernel(feat, edge_index, ln1_g, ln1_b, W_head, W_tail, W_ent, attn,
           g_head, g_tail, ln2_g, ln2_b, W_ff1, b_ff1, W_ff2, b_ff2):
    src = edge_index[0]
    dst = edge_index[1]
    npad = EPAD - E
    sidx1d = jnp.pad(src, (0, npad), constant_values=0)
    didx_s1d = jnp.pad(dst, (0, npad), constant_values=N)
    didx_g1d = jnp.minimum(didx_s1d, N - 1)

    deg2d = _degtc_call(dst.reshape(E, 1))
    degflat = deg2d.T.reshape(128 * 128)[:N]
    deg16 = jnp.broadcast_to(degflat.reshape(N, 1), (N, 16))

    attnrep = attn.reshape(1, D)
    ghrep = g_head.reshape(1, D)
    gtrep = g_tail.reshape(1, D)
    mg = jnp.asarray(_MG)
    fh, fta, fe, ghp, gtp, lg16, msf, mst = _prep_call(
        feat, ln1_g.reshape(1, D), ln1_b.reshape(1, D), W_head, W_tail, W_ent,
        attnrep, ghrep, gtrep, mg, deg16)

    zpad = jnp.zeros((N, 120), F32)
    s_tab = jnp.concatenate([fh, ghp[:, :H], zpad], axis=1)
    t_tab = jnp.concatenate(
        [fta, gtp[:, :H], lg16[:, :1], zpad[:, :119]], axis=1)

    g8 = jnp.sqrt(msf[0, :H]) * jnp.sqrt(mst[0, :H]) * (1.0 / DH)
    g16 = jnp.broadcast_to(
        jnp.concatenate([g8, jnp.zeros((8,), F32)]).reshape(1, 16), (8, 16))

    tag = _edge_ac_kernel(s_tab, t_tab, g16, sidx1d, didx_g1d)

    def one_hop(i, carry):
        f, ri = carry
        hh = _hop_kernel(f, tag, sidx1d, didx_s1d)
        h0 = hh[:N]
        h1 = hh[NACC:NACC + N]
        ri_new = jnp.where(i == 0, _rinv_call(h0, h1), ri)
        f_new = jnp.where(i == 0, fe, _comb_call(h0, h1, ri_new, fe))
        return (f_new, ri_new)
    ones128 = jnp.ones((N, D), F32)
    f, _ = lax.fori_loop(0, HOP + 1, one_hop,
                         (ones128, jnp.zeros((N, D), F32)))

    return _final_call(f, feat, ln2_g.reshape(1, D), ln2_b.reshape(1, D),
                       W_ff1, b_ff1.reshape(1, DFF), W_ff2, b_ff2.reshape(1, D))
